# baseline (device time: 235297 ns/iter reference)
import jax
import jax.numpy as jnp
from jax import lax
from jax.experimental import pallas as pl
from jax.experimental.pallas import tpu as pltpu

XS, YS, ZS = 2, 2, 4
M = 2048
F = 8192
BLK = F // (XS * ZS)
HALF = M // YS
NS = 2
SUB = BLK // NS
MESH = pl.DeviceIdType.MESH


def kernel(x, dy):
    k_per, m = x.shape
    _, f = dy.shape

    c_out = 2 * lax.axis_index("z") + lax.axis_index("x")
    dy_blk = lax.dynamic_slice(dy, (0, c_out * BLK), (k_per, BLK))

    def body(x_ref, dyv, out_ref, pp, pm, ybuf,
             st_sem, y_send, y_recv,
             x_send, x_recv, zr_send, zr_recv, zl_send, zl_recv,
             rl_send, rl_recv, rr_send, rr_recv):
        my_x = lax.axis_index("x")
        my_y = lax.axis_index("y")
        my_z = lax.axis_index("z")
        c = 2 * my_z + my_x
        has_l = my_z > 0
        has_r = my_z < ZS - 1
        xp = (1 - my_x, my_y, my_z)
        yp = (my_x, 1 - my_y, my_z)
        zpd = (my_x, my_y, my_z + 1)
        zmd = (my_x, my_y, my_z - 1)

        def bcols(b, s):
            return pl.ds((2 * b + my_x) * BLK + s * SUB, SUB)

        def sub(s):
            return pl.ds(s * SUB, SUB)

        barrier = pltpu.get_barrier_semaphore()
        for nbr in (xp, yp):
            pl.semaphore_signal(barrier, inc=1, device_id=nbr,
                                device_id_type=MESH)

        @pl.when(has_l)
        def _():
            pl.semaphore_signal(barrier, inc=1, device_id=zmd,
                                device_id_type=MESH)

        @pl.when(has_r)
        def _():
            pl.semaphore_signal(barrier, inc=1, device_id=zpd,
                                device_id_type=MESH)

        def dot_half(rows, s):
            return lax.dot_general(
                x_ref[:, rows], dyv[:, sub(s)],
                dimension_numbers=(((0,), (0,)), ((), ())),
                preferred_element_type=jnp.float32,
            )

        QW = SUB // 2

        def qcols(q):
            return pl.ds(q * QW, QW)

        def dot_q(rows, q):
            return lax.dot_general(
                x_ref[:, rows], dyv[:, qcols(q)],
                dimension_numbers=(((0,), (0,)), ((), ())),
                preferred_element_type=jnp.float32,
            )

        peer_rows = pl.ds((1 - my_y) * HALF, HALF)
        my_rows = pl.ds(my_y * HALF, HALF)

        def y_rdma(q):
            return pltpu.make_async_remote_copy(
                src_ref=pp.at[:, qcols(q)], dst_ref=ybuf.at[:, qcols(q)],
                send_sem=y_send.at[q], recv_sem=y_recv.at[q],
                device_id=yp, device_id_type=MESH,
            )

        def x_sub(b, s, src):
            return pltpu.make_async_remote_copy(
                src_ref=src, dst_ref=out_ref.at[:, bcols(b, s)],
                send_sem=x_send.at[b, s], recv_sem=x_recv.at[b, s],
                device_id=xp, device_id_type=MESH,
            )

        def z_sub(b, s, h, src, sends, recvs, dev):
            return pltpu.make_async_remote_copy(
                src_ref=src, dst_ref=out_ref.at[:, bcols(b, s)],
                send_sem=sends.at[h, s], recv_sem=recvs.at[h, s],
                device_id=dev, device_id_type=MESH,
            )

        def pcols(b, s):
            return pl.ds((2 * b + 1 - my_x) * BLK + s * SUB, SUB)

        def x_recv_wait(b, s):
            pltpu.make_async_remote_copy(
                src_ref=pm.at[:, sub(s)],
                dst_ref=out_ref.at[:, pcols(b, s)],
                send_sem=x_send.at[b, s], recv_sem=x_recv.at[b, s],
                device_id=xp, device_id_type=MESH,
            ).wait_recv()

        def rl_copy(dev):
            cf = pl.ds((2 * (ZS - 1) + 1 - my_x) * BLK, SUB)
            return pltpu.make_async_remote_copy(
                src_ref=out_ref.at[:, cf], dst_ref=out_ref.at[:, cf],
                send_sem=rl_send, recv_sem=rl_recv,
                device_id=dev, device_id_type=MESH,
            )

        def rr_copy(dev):
            cf = pl.ds((1 - my_x) * BLK, SUB)
            return pltpu.make_async_remote_copy(
                src_ref=out_ref.at[:, cf], dst_ref=out_ref.at[:, cf],
                send_sem=rr_send, recv_sem=rr_recv,
                device_id=dev, device_id_type=MESH,
            )

        def launch(s):
            pltpu.make_async_copy(
                pm.at[:, sub(s)], out_ref.at[:, bcols(my_z, s)],
                st_sem.at[s]).start()
            x_sub(my_z, s, pm.at[:, sub(s)]).start()

            @pl.when(has_r)
            def _():
                z_sub(my_z, s, 0, pm.at[:, sub(s)],
                      zr_send, zr_recv, zpd).start()

            @pl.when(has_l)
            def _():
                z_sub(my_z, s, 0, pm.at[:, sub(s)],
                      zl_send, zl_recv, zmd).start()

        pp[:, qcols(0)] = dot_q(peer_rows, 0)
        n_nbrs = 2 + has_l.astype(jnp.int32) + has_r.astype(jnp.int32)
        pl.semaphore_wait(barrier, n_nbrs)
        y_rdma(0).start()
        pp[:, qcols(1)] = dot_q(peer_rows, 1)
        y_rdma(1).start()
        pm[:, sub(0)] = dot_half(my_rows, 0)
        y_rdma(0).wait()
        y_rdma(1).wait()
        pm[:, sub(0)] = pm[:, sub(0)] + ybuf[:, sub(0)]
        launch(0)

        pp[:, qcols(2)] = dot_q(peer_rows, 2)
        y_rdma(2).start()
        pp[:, qcols(3)] = dot_q(peer_rows, 3)
        y_rdma(3).start()
        pm[:, sub(1)] = dot_half(my_rows, 1)
        y_rdma(2).wait()
        y_rdma(3).wait()
        pm[:, sub(1)] = pm[:, sub(1)] + ybuf[:, sub(1)]
        launch(1)

        for h in range(ZS - 1):
            rb = my_z - 1 - h
            lb = my_z + 1 + h

            @pl.when(has_l & (rb >= 0))
            def _(rb=rb, h=h):
                for s in range(NS):
                    z_sub(rb, s, h, pm.at[:, sub(s)],
                          zr_send, zr_recv, zmd).wait_recv()
                    src = out_ref.at[:, bcols(rb, s)]
                    if h + 1 < ZS - 1:
                        @pl.when(has_r)
                        def _(s=s, src=src):
                            z_sub(rb, s, h + 1, src,
                                  zr_send, zr_recv, zpd).start()
                    if s == 0:
                        @pl.when(rb != 0)
                        def _(src=src):
                            x_sub(rb, 0, src).start()
                    else:
                        x_sub(rb, s, src).start()

            @pl.when(has_r & (lb <= ZS - 1))
            def _(lb=lb, h=h):
                for s in range(NS):
                    z_sub(lb, s, h, pm.at[:, sub(s)],
                          zl_send, zl_recv, zpd).wait_recv()
                    src = out_ref.at[:, bcols(lb, s)]
                    if h + 1 < ZS - 1:
                        @pl.when(has_l)
                        def _(s=s, src=src):
                            z_sub(lb, s, h + 1, src,
                                  zl_send, zl_recv, zmd).start()
                    if s == 0:
                        @pl.when(lb != ZS - 1)
                        def _(src=src):
                            x_sub(lb, 0, src).start()
                    else:
                        x_sub(lb, s, src).start()

            if h == 0:
                @pl.when(my_z == ZS - 1)
                def _():
                    x_recv_wait(ZS - 1, 0)
                    rl_copy(zmd).start()

                @pl.when(my_z == 0)
                def _():
                    x_recv_wait(0, 0)
                    rr_copy(zpd).start()

                @pl.when(my_z == 2)
                def _():
                    rl_copy(zmd).wait_recv()
                    rl_copy(zmd).start()

                @pl.when(my_z == 1)
                def _():
                    rr_copy(zpd).wait_recv()
                    rr_copy(zpd).start()
            elif h == 1:
                @pl.when(my_z == 1)
                def _():
                    rl_copy(zmd).wait_recv()
                    rl_copy(zmd).start()

                @pl.when(my_z == 2)
                def _():
                    rr_copy(zpd).wait_recv()
                    rr_copy(zpd).start()
            else:
                @pl.when(my_z == 0)
                def _():
                    rl_copy(zmd).wait_recv()

                @pl.when(my_z == ZS - 1)
                def _():
                    rr_copy(zpd).wait_recv()

        for s in range(NS):
            pltpu.make_async_copy(
                pm.at[:, sub(s)], out_ref.at[:, bcols(my_z, s)],
                st_sem.at[s]).wait()
        for b in range(ZS):
            for s in range(NS):
                if s == 0 and b in (0, ZS - 1):
                    @pl.when(my_z == b)
                    def _(b=b):
                        x_sub(b, 0, pm.at[:, sub(0)]).wait_send()
                    continue
                x_sub(b, s, pm.at[:, sub(s)]).wait_send()
                pltpu.make_async_remote_copy(
                    src_ref=pm.at[:, sub(s)],
                    dst_ref=out_ref.at[:, pcols(b, s)],
                    send_sem=x_send.at[b, s], recv_sem=x_recv.at[b, s],
                    device_id=xp, device_id_type=MESH,
                ).wait_recv()
        @pl.when(my_z >= 1)
        def _():
            rl_copy(zmd).wait_send()

        @pl.when(my_z <= ZS - 2)
        def _():
            rr_copy(zpd).wait_send()
        for h in range(ZS - 1):
            @pl.when(has_r & (my_z - h >= 0))
            def _(h=h):
                for s in range(NS):
                    z_sub(0, s, h, pm.at[:, sub(s)],
                          zr_send, zr_recv, zpd).wait_send()

            @pl.when(has_l & (my_z + h <= ZS - 1))
            def _(h=h):
                for s in range(NS):
                    z_sub(0, s, h, pm.at[:, sub(s)],
                          zl_send, zl_recv, zmd).wait_send()

    return pl.pallas_call(
        body,
        out_shape=jax.ShapeDtypeStruct((HALF, f), jnp.float32),
        in_specs=[
            pl.BlockSpec(memory_space=pltpu.MemorySpace.VMEM),
            pl.BlockSpec(memory_space=pltpu.MemorySpace.VMEM),
        ],
        out_specs=pl.BlockSpec(memory_space=pltpu.MemorySpace.HBM),
        scratch_shapes=[
            pltpu.VMEM((HALF, BLK), jnp.float32),
            pltpu.VMEM((HALF, BLK), jnp.float32),
            pltpu.VMEM((HALF, BLK), jnp.float32),
            pltpu.SemaphoreType.DMA((NS,)),
            pltpu.SemaphoreType.DMA((4,)),
            pltpu.SemaphoreType.DMA((4,)),
            pltpu.SemaphoreType.DMA((ZS, NS)),
            pltpu.SemaphoreType.DMA((ZS, NS)),
            pltpu.SemaphoreType.DMA((ZS - 1, NS)),
            pltpu.SemaphoreType.DMA((ZS - 1, NS)),
            pltpu.SemaphoreType.DMA((ZS - 1, NS)),
            pltpu.SemaphoreType.DMA((ZS - 1, NS)),
            pltpu.SemaphoreType.DMA,
            pltpu.SemaphoreType.DMA,
            pltpu.SemaphoreType.DMA,
            pltpu.SemaphoreType.DMA,
        ],
        compiler_params=pltpu.CompilerParams(
            collective_id=0,
            vmem_limit_bytes=60 * 1024 * 1024,
        ),
    )(x, dy_blk)


# device time: 228000 ns/iter; 1.0320x vs baseline; 1.0320x over previous
import jax
import jax.numpy as jnp
from jax import lax
from jax.experimental import pallas as pl
from jax.experimental.pallas import tpu as pltpu

XS, YS, ZS = 2, 2, 4
M = 2048
F = 8192
BLK = F // (XS * ZS)
HALF = M // YS
NS = 2
SUB = BLK // NS
MESH = pl.DeviceIdType.MESH


def kernel(x, dy):
    k_per, m = x.shape
    _, f = dy.shape

    def body(x_ref, dy_ref, out_ref, dyv, pp, pm, ybuf,
             dy_sem, st_sem, y_send, y_recv,
             x_send, x_recv, zr_send, zr_recv, zl_send, zl_recv,
             rl_send, rl_recv, rr_send, rr_recv):
        my_x = lax.axis_index("x")
        my_y = lax.axis_index("y")
        my_z = lax.axis_index("z")
        c = 2 * my_z + my_x
        has_l = my_z > 0
        has_r = my_z < ZS - 1
        xp = (1 - my_x, my_y, my_z)
        yp = (my_x, 1 - my_y, my_z)
        zpd = (my_x, my_y, my_z + 1)
        zmd = (my_x, my_y, my_z - 1)

        def bcols(b, s):
            return pl.ds((2 * b + my_x) * BLK + s * SUB, SUB)

        def sub(s):
            return pl.ds(s * SUB, SUB)

        barrier = pltpu.get_barrier_semaphore()
        for nbr in (xp, yp):
            pl.semaphore_signal(barrier, inc=1, device_id=nbr,
                                device_id_type=MESH)

        @pl.when(has_l)
        def _():
            pl.semaphore_signal(barrier, inc=1, device_id=zmd,
                                device_id_type=MESH)

        @pl.when(has_r)
        def _():
            pl.semaphore_signal(barrier, inc=1, device_id=zpd,
                                device_id_type=MESH)

        cps = [pltpu.make_async_copy(
                   dy_ref.at[:, pl.ds(c * BLK + s * SUB, SUB)],
                   dyv.at[:, sub(s)], dy_sem.at[s]) for s in range(NS)]
        cps[0].start()
        cps[1].start()

        def dot_half(rows, s):
            return lax.dot_general(
                x_ref[:, rows], dyv[:, sub(s)],
                dimension_numbers=(((0,), (0,)), ((), ())),
                preferred_element_type=jnp.float32,
            )

        QW = SUB // 2

        def qcols(q):
            return pl.ds(q * QW, QW)

        def dot_q(rows, q):
            return lax.dot_general(
                x_ref[:, rows], dyv[:, qcols(q)],
                dimension_numbers=(((0,), (0,)), ((), ())),
                preferred_element_type=jnp.float32,
            )

        peer_rows = pl.ds((1 - my_y) * HALF, HALF)
        my_rows = pl.ds(my_y * HALF, HALF)

        def y_rdma(q):
            return pltpu.make_async_remote_copy(
                src_ref=pp.at[:, qcols(q)], dst_ref=ybuf.at[:, qcols(q)],
                send_sem=y_send.at[q], recv_sem=y_recv.at[q],
                device_id=yp, device_id_type=MESH,
            )

        def x_sub(b, s, src):
            return pltpu.make_async_remote_copy(
                src_ref=src, dst_ref=out_ref.at[:, bcols(b, s)],
                send_sem=x_send.at[b, s], recv_sem=x_recv.at[b, s],
                device_id=xp, device_id_type=MESH,
            )

        def z_sub(b, s, h, src, sends, recvs, dev):
            return pltpu.make_async_remote_copy(
                src_ref=src, dst_ref=out_ref.at[:, bcols(b, s)],
                send_sem=sends.at[h, s], recv_sem=recvs.at[h, s],
                device_id=dev, device_id_type=MESH,
            )

        def pcols(b, s):
            return pl.ds((2 * b + 1 - my_x) * BLK + s * SUB, SUB)

        def x_recv_wait(b, s):
            pltpu.make_async_remote_copy(
                src_ref=pm.at[:, sub(s)],
                dst_ref=out_ref.at[:, pcols(b, s)],
                send_sem=x_send.at[b, s], recv_sem=x_recv.at[b, s],
                device_id=xp, device_id_type=MESH,
            ).wait_recv()

        def rl_copy(dev):
            cf = pl.ds((2 * (ZS - 1) + 1 - my_x) * BLK, SUB)
            return pltpu.make_async_remote_copy(
                src_ref=out_ref.at[:, cf], dst_ref=out_ref.at[:, cf],
                send_sem=rl_send, recv_sem=rl_recv,
                device_id=dev, device_id_type=MESH,
            )

        def rr_copy(dev):
            cf = pl.ds((1 - my_x) * BLK, SUB)
            return pltpu.make_async_remote_copy(
                src_ref=out_ref.at[:, cf], dst_ref=out_ref.at[:, cf],
                send_sem=rr_send, recv_sem=rr_recv,
                device_id=dev, device_id_type=MESH,
            )

        def launch(s):
            pltpu.make_async_copy(
                pm.at[:, sub(s)], out_ref.at[:, bcols(my_z, s)],
                st_sem.at[s]).start()
            x_sub(my_z, s, pm.at[:, sub(s)]).start()

            @pl.when(has_r)
            def _():
                z_sub(my_z, s, 0, pm.at[:, sub(s)],
                      zr_send, zr_recv, zpd).start()

            @pl.when(has_l)
            def _():
                z_sub(my_z, s, 0, pm.at[:, sub(s)],
                      zl_send, zl_recv, zmd).start()

        cps[0].wait()
        pp[:, qcols(0)] = dot_q(peer_rows, 0)
        n_nbrs = 2 + has_l.astype(jnp.int32) + has_r.astype(jnp.int32)
        pl.semaphore_wait(barrier, n_nbrs)
        y_rdma(0).start()
        pp[:, qcols(1)] = dot_q(peer_rows, 1)
        y_rdma(1).start()
        pm[:, sub(0)] = dot_half(my_rows, 0)
        y_rdma(0).wait()
        y_rdma(1).wait()
        pm[:, sub(0)] = pm[:, sub(0)] + ybuf[:, sub(0)]
        launch(0)

        cps[1].wait()
        pp[:, qcols(2)] = dot_q(peer_rows, 2)
        y_rdma(2).start()
        pp[:, qcols(3)] = dot_q(peer_rows, 3)
        y_rdma(3).start()
        pm[:, sub(1)] = dot_half(my_rows, 1)
        y_rdma(2).wait()
        y_rdma(3).wait()
        pm[:, sub(1)] = pm[:, sub(1)] + ybuf[:, sub(1)]
        launch(1)

        for h in range(ZS - 1):
            rb = my_z - 1 - h
            lb = my_z + 1 + h

            @pl.when(has_l & (rb >= 0))
            def _(rb=rb, h=h):
                for s in range(NS):
                    z_sub(rb, s, h, pm.at[:, sub(s)],
                          zr_send, zr_recv, zmd).wait_recv()
                    src = out_ref.at[:, bcols(rb, s)]
                    if h + 1 < ZS - 1:
                        @pl.when(has_r)
                        def _(s=s, src=src):
                            z_sub(rb, s, h + 1, src,
                                  zr_send, zr_recv, zpd).start()
                    if s == 0:
                        @pl.when(rb != 0)
                        def _(src=src):
                            x_sub(rb, 0, src).start()
                    else:
                        x_sub(rb, s, src).start()

            @pl.when(has_r & (lb <= ZS - 1))
            def _(lb=lb, h=h):
                for s in range(NS):
                    z_sub(lb, s, h, pm.at[:, sub(s)],
                          zl_send, zl_recv, zpd).wait_recv()
                    src = out_ref.at[:, bcols(lb, s)]
                    if h + 1 < ZS - 1:
                        @pl.when(has_l)
                        def _(s=s, src=src):
                            z_sub(lb, s, h + 1, src,
                                  zl_send, zl_recv, zmd).start()
                    if s == 0:
                        @pl.when(lb != ZS - 1)
                        def _(src=src):
                            x_sub(lb, 0, src).start()
                    else:
                        x_sub(lb, s, src).start()

            if h == 0:
                @pl.when(my_z == ZS - 1)
                def _():
                    x_recv_wait(ZS - 1, 0)
                    rl_copy(zmd).start()

                @pl.when(my_z == 0)
                def _():
                    x_recv_wait(0, 0)
                    rr_copy(zpd).start()

                @pl.when(my_z == 2)
                def _():
                    rl_copy(zmd).wait_recv()
                    rl_copy(zmd).start()

                @pl.when(my_z == 1)
                def _():
                    rr_copy(zpd).wait_recv()
                    rr_copy(zpd).start()
            elif h == 1:
                @pl.when(my_z == 1)
                def _():
                    rl_copy(zmd).wait_recv()
                    rl_copy(zmd).start()

                @pl.when(my_z == 2)
                def _():
                    rr_copy(zpd).wait_recv()
                    rr_copy(zpd).start()
            else:
                @pl.when(my_z == 0)
                def _():
                    rl_copy(zmd).wait_recv()

                @pl.when(my_z == ZS - 1)
                def _():
                    rr_copy(zpd).wait_recv()

        for s in range(NS):
            pltpu.make_async_copy(
                pm.at[:, sub(s)], out_ref.at[:, bcols(my_z, s)],
                st_sem.at[s]).wait()
        for b in range(ZS):
            for s in range(NS):
                if s == 0 and b in (0, ZS - 1):
                    @pl.when(my_z == b)
                    def _(b=b):
                        x_sub(b, 0, pm.at[:, sub(0)]).wait_send()
                    continue
                x_sub(b, s, pm.at[:, sub(s)]).wait_send()
                pltpu.make_async_remote_copy(
                    src_ref=pm.at[:, sub(s)],
                    dst_ref=out_ref.at[:, pcols(b, s)],
                    send_sem=x_send.at[b, s], recv_sem=x_recv.at[b, s],
                    device_id=xp, device_id_type=MESH,
                ).wait_recv()
        @pl.when(my_z >= 1)
        def _():
            rl_copy(zmd).wait_send()

        @pl.when(my_z <= ZS - 2)
        def _():
            rr_copy(zpd).wait_send()
        for h in range(ZS - 1):
            @pl.when(has_r & (my_z - h >= 0))
            def _(h=h):
                for s in range(NS):
                    z_sub(0, s, h, pm.at[:, sub(s)],
                          zr_send, zr_recv, zpd).wait_send()

            @pl.when(has_l & (my_z + h <= ZS - 1))
            def _(h=h):
                for s in range(NS):
                    z_sub(0, s, h, pm.at[:, sub(s)],
                          zl_send, zl_recv, zmd).wait_send()

    return pl.pallas_call(
        body,
        out_shape=jax.ShapeDtypeStruct((HALF, f), jnp.float32),
        in_specs=[
            pl.BlockSpec(memory_space=pltpu.MemorySpace.VMEM),
            pl.BlockSpec(memory_space=pltpu.MemorySpace.HBM),
        ],
        out_specs=pl.BlockSpec(memory_space=pltpu.MemorySpace.HBM),
        scratch_shapes=[
            pltpu.VMEM((k_per, BLK), jnp.float32),
            pltpu.VMEM((HALF, BLK), jnp.float32),
            pltpu.VMEM((HALF, BLK), jnp.float32),
            pltpu.VMEM((HALF, BLK), jnp.float32),
            pltpu.SemaphoreType.DMA((NS,)),
            pltpu.SemaphoreType.DMA((NS,)),
            pltpu.SemaphoreType.DMA((4,)),
            pltpu.SemaphoreType.DMA((4,)),
            pltpu.SemaphoreType.DMA((ZS, NS)),
            pltpu.SemaphoreType.DMA((ZS, NS)),
            pltpu.SemaphoreType.DMA((ZS - 1, NS)),
            pltpu.SemaphoreType.DMA((ZS - 1, NS)),
            pltpu.SemaphoreType.DMA((ZS - 1, NS)),
            pltpu.SemaphoreType.DMA((ZS - 1, NS)),
            pltpu.SemaphoreType.DMA,
            pltpu.SemaphoreType.DMA,
            pltpu.SemaphoreType.DMA,
            pltpu.SemaphoreType.DMA,
        ],
        compiler_params=pltpu.CompilerParams(
            collective_id=0,
            vmem_limit_bytes=60 * 1024 * 1024,
        ),
    )(x, dy)
